# baseline (speedup 1.0000x reference)
import functools

import jax
import jax.numpy as jnp
from jax import lax
from jax.experimental import pallas as pl
from jax.experimental.pallas import tpu as pltpu
from jax.experimental.pallas import tpu_sc as plsc

ENT_NUM = 1000000
REL_NUM = 1000
DIM = 64
B = 4096
NEG = 200

_INFO = plsc.get_sparse_core_info()
NC = _INFO.num_cores
NS = _INFO.num_subcores
NW = NC * NS
CH = 128
K = 8
BLK = K * CH

NEG_TOT = B * NEG
NEG_PER_W = NEG_TOT // NW
NEG_BLKS = NEG_PER_W // BLK
HT_TOT = 2 * B
HT_PER_W = HT_TOT // NW
REL_PER_W = B // NW


def _sc_gather(ent_hbm, rel_hbm, negidx_hbm, htidx_hbm, relidx_hbm,
               neg_out, ht_out, rel_out, idx_v, rows_v, sem):
    wid = lax.axis_index("s") * NC + lax.axis_index("c")

    neg_row0 = wid * (NEG_PER_W // CH)

    def blk_body(b, _):
        r0 = neg_row0 + b * K
        pltpu.sync_copy(negidx_hbm.at[pl.ds(r0, K)], idx_v)
        descs = [
            pltpu.async_copy(ent_hbm.at[idx_v.at[j]],
                             rows_v.at[pl.ds(j * CH, CH)], sem)
            for j in range(K)
        ]
        for d in descs:
            d.wait()
        pltpu.sync_copy(rows_v,
                        neg_out.at[pl.ds(wid * NEG_PER_W + b * BLK, BLK)])
        return _

    lax.fori_loop(0, NEG_BLKS, blk_body, 0)

    ht_r0 = wid * (HT_PER_W // CH)
    pltpu.sync_copy(htidx_hbm.at[pl.ds(ht_r0, 2)], idx_v.at[pl.ds(0, 2)])
    d0 = pltpu.async_copy(ent_hbm.at[idx_v.at[0]],
                          rows_v.at[pl.ds(0, CH)], sem)
    d1 = pltpu.async_copy(ent_hbm.at[idx_v.at[1]],
                          rows_v.at[pl.ds(CH, CH)], sem)
    d0.wait()
    d1.wait()
    pltpu.sync_copy(rows_v.at[pl.ds(0, HT_PER_W)],
                    ht_out.at[pl.ds(wid * HT_PER_W, HT_PER_W)])

    pltpu.sync_copy(relidx_hbm.at[pl.ds(wid, 1)], idx_v.at[pl.ds(0, 1)])
    pltpu.async_copy(rel_hbm.at[idx_v.at[0]],
                     rows_v.at[pl.ds(0, CH)], sem).wait()
    pltpu.sync_copy(rows_v.at[pl.ds(0, REL_PER_W)],
                    rel_out.at[pl.ds(wid * REL_PER_W, REL_PER_W)])


@jax.jit
def _run(pos_sample, neg_sample, ent_embd, rel_embd):
    negidx = neg_sample.reshape(NEG_TOT // CH, CH)
    htidx = jnp.concatenate(
        [pos_sample[:, 0], pos_sample[:, 2]]).reshape(HT_TOT // CH, CH)
    relidx = pos_sample[:, 1].reshape(B // CH, CH)

    mesh = plsc.VectorSubcoreMesh(core_axis_name="c", subcore_axis_name="s")
    neg_rows, ht_rows, rel_rows = pl.kernel(
        _sc_gather,
        out_type=[
            jax.ShapeDtypeStruct((NEG_TOT, DIM), jnp.float32),
            jax.ShapeDtypeStruct((HT_TOT, DIM), jnp.float32),
            jax.ShapeDtypeStruct((B, DIM), jnp.float32),
        ],
        mesh=mesh,
        scratch_types=[
            pltpu.VMEM((K, CH), jnp.int32),
            pltpu.VMEM((BLK, DIM), jnp.float32),
            pltpu.SemaphoreType.DMA,
        ],
        compiler_params=pltpu.CompilerParams(use_tc_tiling_on_sc=False),
        name="kge_embed_gather",
    )(ent_embd, rel_embd, negidx, htidx, relidx)

    head = ht_rows[:B, None, :]
    tail = ht_rows[B:, None, :]
    relation = rel_rows[:, None, :]
    neg = neg_rows.reshape(B, NEG, DIM)
    return head, relation, tail, neg


def kernel(pos_sample, neg_sample, ent_embd, rel_embd):
    return _run(pos_sample, neg_sample, ent_embd, rel_embd)


# speedup vs baseline: 1.0147x; 1.0147x over previous
import jax
import jax.numpy as jnp
from jax import lax
from jax.experimental import pallas as pl
from jax.experimental.pallas import tpu as pltpu
from jax.experimental.pallas import tpu_sc as plsc

ENT_NUM = 1000000
REL_NUM = 1000
DIM = 64
B = 4096
NEG = 200

_INFO = plsc.get_sparse_core_info()
NC = _INFO.num_cores
NS = _INFO.num_subcores
NW = NC * NS
CH = 128
K = 4
BLK = K * CH

NEG_TOT = B * NEG
NEG_PER_W = NEG_TOT // NW
NEG_ROWS_W = NEG_PER_W // CH
NB = NEG_PER_W // BLK
HT_TOT = 2 * B
HT_PER_W = HT_TOT // NW
REL_PER_W = B // NW


def _sc_gather(ent_hbm, rel_hbm, negidx_hbm, htidx_hbm, relidx_hbm,
               neg_out, ht_out, rel_out, idx_v, rows_v, gsem, wsem):
    wid = lax.axis_index("s") * NC + lax.axis_index("c")
    out0 = wid * NEG_PER_W

    pltpu.sync_copy(negidx_hbm.at[pl.ds(wid * NEG_ROWS_W, NEG_ROWS_W)], idx_v)

    def fire_gathers(b, p):
        for j in range(K):
            pltpu.async_copy(ent_hbm.at[idx_v.at[b * K + j]],
                             rows_v.at[p, pl.ds(j * CH, CH)], gsem)

    def drain(p, sem):
        pltpu.make_async_copy(neg_out.at[pl.ds(out0, BLK)],
                              rows_v.at[p], sem).wait()

    fire_gathers(0, 0)

    def blk_body(g, carry):
        p = lax.rem(g, 2)
        drain(p, gsem)
        pltpu.async_copy(rows_v.at[p],
                         neg_out.at[pl.ds(out0 + g * BLK, BLK)], wsem)

        @pl.when(g + 1 < NB)
        def _():
            @pl.when(g >= 1)
            def _():
                drain(1 - p, wsem)
            fire_gathers(g + 1, 1 - p)

        return carry

    lax.fori_loop(0, NB, blk_body, 0)

    drain(lax.rem(NB, 2), wsem)
    drain(lax.rem(NB + 1, 2), wsem)

    pltpu.sync_copy(htidx_hbm.at[pl.ds(wid * (HT_PER_W // CH), 2)],
                    idx_v.at[pl.ds(0, 2)])
    d0 = pltpu.async_copy(ent_hbm.at[idx_v.at[0]],
                          rows_v.at[0, pl.ds(0, CH)], gsem)
    d1 = pltpu.async_copy(ent_hbm.at[idx_v.at[1]],
                          rows_v.at[0, pl.ds(CH, CH)], gsem)
    d0.wait()
    d1.wait()
    pltpu.sync_copy(rows_v.at[0, pl.ds(0, HT_PER_W)],
                    ht_out.at[pl.ds(wid * HT_PER_W, HT_PER_W)])

    pltpu.sync_copy(relidx_hbm.at[pl.ds(wid, 1)], idx_v.at[pl.ds(0, 1)])
    pltpu.async_copy(rel_hbm.at[idx_v.at[0]],
                     rows_v.at[0, pl.ds(0, CH)], gsem).wait()
    pltpu.sync_copy(rows_v.at[0, pl.ds(0, REL_PER_W)],
                    rel_out.at[pl.ds(wid * REL_PER_W, REL_PER_W)])


@jax.jit
def _run(pos_sample, neg_sample, ent_embd, rel_embd):
    negidx = neg_sample.reshape(NEG_TOT // CH, CH)
    htidx = jnp.concatenate(
        [pos_sample[:, 0], pos_sample[:, 2]]).reshape(HT_TOT // CH, CH)
    relidx = pos_sample[:, 1].reshape(B // CH, CH)

    mesh = plsc.VectorSubcoreMesh(core_axis_name="c", subcore_axis_name="s")
    neg_rows, ht_rows, rel_rows = pl.kernel(
        _sc_gather,
        out_type=[
            jax.ShapeDtypeStruct((NEG_TOT, DIM), jnp.float32),
            jax.ShapeDtypeStruct((HT_TOT, DIM), jnp.float32),
            jax.ShapeDtypeStruct((B, DIM), jnp.float32),
        ],
        mesh=mesh,
        scratch_types=[
            pltpu.VMEM((NEG_ROWS_W, CH), jnp.int32),
            pltpu.VMEM((2, BLK, DIM), jnp.float32),
            pltpu.SemaphoreType.DMA,
            pltpu.SemaphoreType.DMA,
        ],
        compiler_params=pltpu.CompilerParams(use_tc_tiling_on_sc=False),
        name="kge_embed_gather",
    )(ent_embd, rel_embd, negidx, htidx, relidx)

    head = ht_rows[:B, None, :]
    tail = ht_rows[B:, None, :]
    relation = rel_rows[:, None, :]
    neg = neg_rows.reshape(B, NEG, DIM)
    return head, relation, tail, neg


def kernel(pos_sample, neg_sample, ent_embd, rel_embd):
    return _run(pos_sample, neg_sample, ent_embd, rel_embd)
